# Initial kernel scaffold; baseline (speedup 1.0000x reference)
import jax
import jax.numpy as jnp
from jax.experimental import pallas as pl


def kernel(shape_id, emb_weight):
    raise NotImplementedError("write your pallas kernel here")


# speedup vs baseline: 9.2959x; 9.2959x over previous
import functools

import jax
import jax.numpy as jnp
from jax import lax
from jax.experimental import pallas as pl
from jax.experimental.pallas import tpu as pltpu
from jax.experimental.pallas import tpu_sc as plsc

_CHUNK = 128
_NBUF = 4


@functools.cache
def _build(num_rows: int, vocab: int, dim: int):
  info = plsc.get_sparse_core_info()
  nc, ns = info.num_cores, info.num_subcores
  nw = nc * ns
  assert num_rows % (nw * _CHUNK) == 0
  chunks_per_w = num_rows // (nw * _CHUNK)
  assert chunks_per_w % _NBUF == 0
  n_outer = chunks_per_w // _NBUF
  rows_per_w = chunks_per_w * _CHUNK

  mesh = plsc.VectorSubcoreMesh(core_axis_name="c", subcore_axis_name="s")

  @functools.partial(
      pl.kernel,
      out_type=jax.ShapeDtypeStruct((num_rows, dim), jnp.float32),
      mesh=mesh,
      scratch_types=[
          pltpu.VMEM((chunks_per_w, _CHUNK), jnp.int32),
          [pltpu.VMEM((_CHUNK, dim), jnp.float32) for _ in range(_NBUF)],
          [pltpu.SemaphoreType.DMA for _ in range(_NBUF)],
      ],
  )
  def gather_kernel(table_hbm, idx_hbm, out_hbm, idx_v, bufs, gsems):
    wid = lax.axis_index("s") * nc + lax.axis_index("c")
    chunk0 = wid * chunks_per_w
    row0 = wid * rows_per_w

    pltpu.sync_copy(idx_hbm.at[pl.ds(chunk0, chunks_per_w)], idx_v)

    def start_gather(c, b):
      pltpu.async_copy(table_hbm.at[idx_v.at[c]], bufs[b], gsems[b])

    def wait_gather(c, b):
      pltpu.make_async_copy(table_hbm.at[idx_v.at[c]], bufs[b], gsems[b]).wait()

    def write_out(c, b):
      pltpu.sync_copy(bufs[b], out_hbm.at[pl.ds(row0 + c * _CHUNK, _CHUNK)])

    for b in range(_NBUF):
      start_gather(b, b)

    @pl.loop(0, n_outer - 1)
    def _(i):
      for b in range(_NBUF):
        c = i * _NBUF + b
        wait_gather(c, b)
        write_out(c, b)
        start_gather(c + _NBUF, b)

    for b in range(_NBUF):
      c = (n_outer - 1) * _NBUF + b
      wait_gather(c, b)
      write_out(c, b)

  return gather_kernel


def kernel(shape_id, emb_weight):
  batch, seq = shape_id.shape
  vocab, dim = emb_weight.shape
  num_rows = batch * seq
  idx = shape_id.astype(jnp.int32).reshape(num_rows // _CHUNK, _CHUNK)
  out = _build(num_rows, vocab, dim)(emb_weight, idx)
  return out.reshape(batch, seq, dim)
